# Initial kernel scaffold; baseline (speedup 1.0000x reference)
import jax
import jax.numpy as jnp
from jax.experimental import pallas as pl


def kernel(tokens, W_E):
    raise NotImplementedError("write your pallas kernel here")


# speedup vs baseline: 1.5398x; 1.5398x over previous
import functools

import jax
import jax.numpy as jnp
from jax import lax
from jax.experimental import pallas as pl
from jax.experimental.pallas import tpu as pltpu
from jax.experimental.pallas import tpu_sc as plsc

D_MODEL = 1024
B_TOTAL = 4 * 4096

_NC, _NS = 2, 16
_NW = _NC * _NS
B_PER_W = B_TOTAL // _NW
CHUNK = 64
NCHUNK = B_PER_W // CHUNK


_mesh = plsc.VectorSubcoreMesh(core_axis_name="c", subcore_axis_name="s")


@functools.partial(
    pl.kernel,
    out_type=jax.ShapeDtypeStruct((B_TOTAL, D_MODEL), jnp.float32),
    mesh=_mesh,
    scratch_types=[
        pltpu.VMEM((B_PER_W,), jnp.int32),
        pltpu.VMEM((CHUNK, D_MODEL), jnp.float32),
        pltpu.SemaphoreType.DMA,
    ],
)
def _embed_sc(table_hbm, idx_hbm, out_hbm, idx_v, rows_v, sem):
    wid = lax.axis_index("s") * _NC + lax.axis_index("c")
    base = wid * B_PER_W
    pltpu.sync_copy(idx_hbm.at[pl.ds(base, B_PER_W)], idx_v)
    for c in range(NCHUNK):
        pltpu.async_copy(
            table_hbm.at[idx_v.at[pl.ds(c * CHUNK, CHUNK)]], rows_v, sem
        ).wait()
        pltpu.sync_copy(rows_v, out_hbm.at[pl.ds(base + c * CHUNK, CHUNK)])


def kernel(tokens, W_E):
    idx = tokens.reshape(-1).astype(jnp.int32)
    out = _embed_sc(W_E, idx)
    return out.reshape(tokens.shape + (W_E.shape[1],))


# speedup vs baseline: 1.6604x; 1.0783x over previous
import functools

import jax
import jax.numpy as jnp
from jax import lax
from jax.experimental import pallas as pl
from jax.experimental.pallas import tpu as pltpu
from jax.experimental.pallas import tpu_sc as plsc

D_MODEL = 1024
B_TOTAL = 4 * 4096

_NC, _NS = 2, 16
_NW = _NC * _NS
B_PER_W = B_TOTAL // _NW
CHUNK = 32
NCHUNK = B_PER_W // CHUNK
NBUF = 3


_mesh = plsc.VectorSubcoreMesh(core_axis_name="c", subcore_axis_name="s")


@functools.partial(
    pl.kernel,
    out_type=jax.ShapeDtypeStruct((B_TOTAL, D_MODEL), jnp.float32),
    mesh=_mesh,
    scratch_types=[
        pltpu.VMEM((B_PER_W,), jnp.int32),
        pltpu.VMEM((NBUF, CHUNK, D_MODEL), jnp.float32),
        [pltpu.SemaphoreType.DMA] * NBUF,
        [pltpu.SemaphoreType.DMA] * NBUF,
    ],
)
def _embed_sc(table_hbm, idx_hbm, out_hbm, idx_v, ring, sems_in, sems_out):
    wid = lax.axis_index("s") * _NC + lax.axis_index("c")
    base = wid * B_PER_W
    pltpu.sync_copy(idx_hbm.at[pl.ds(base, B_PER_W)], idx_v)

    def gather(c, b):
        return pltpu.async_copy(
            table_hbm.at[idx_v.at[pl.ds(c * CHUNK, CHUNK)]],
            ring.at[b],
            sems_in[b],
        )

    def writeback(c, b):
        return pltpu.async_copy(
            ring.at[b], out_hbm.at[pl.ds(base + c * CHUNK, CHUNK)], sems_out[b]
        )

    h_in = [gather(b, b) for b in range(NBUF)]
    h_out = [None] * NBUF
    for c in range(NCHUNK):
        b = c % NBUF
        h_in[b].wait()
        h_out[b] = writeback(c, b)
        nxt = c + NBUF
        if nxt < NCHUNK:
            h_out[b].wait()
            h_in[b] = gather(nxt, b)
    for c in range(NCHUNK - NBUF, NCHUNK):
        h_out[c % NBUF].wait()


def kernel(tokens, W_E):
    idx = tokens.reshape(-1).astype(jnp.int32)
    out = _embed_sc(W_E, idx)
    return out.reshape(tokens.shape + (W_E.shape[1],))
